# baseline (speedup 1.0000x reference)
import jax
import jax.numpy as jnp
from jax import lax
from jax.experimental import pallas as pl
from jax.experimental.pallas import tpu as pltpu

_TN = 512


def _fused_body(x_ref, w_ref, c_ref,
                xr_ref, xp_ref, mr_ref, mp_ref, lab_ref,
                c2_ref, caug_ref):
    M, K, d = c_ref.shape
    TN = x_ref.shape[0]

    @pl.when(pl.program_id(0) == 0)
    def _init():
        for m in range(M):
            cm = c_ref[m]
            c2_ref[m:m + 1, :] = jnp.sum(cm * cm, axis=1)[None, :]
            caug_ref[m] = jnp.concatenate(
                [cm, jnp.ones((K, 1), jnp.float32)], axis=1)

    x = x_ref[...]
    w = w_ref[...]
    y = lax.dot_general(x, w, (((1,), (0,)), ((), ())),
                        preferred_element_type=jnp.float32)
    xp_ref[...] = y
    mp_ref[...] = lax.dot_general(y, w, (((1,), (1,)), ((), ())),
                                  preferred_element_type=jnp.float32)
    yr_cols = []
    for m in range(M):
        xm = y[:, m * d:(m + 1) * d]
        xs = xm + xm
        cm = c_ref[m]
        xc2 = lax.dot_general(xs, cm, (((1,), (1,)), ((), ())),
                              preferred_element_type=jnp.float32)
        g = xc2 - c2_ref[m:m + 1, :]
        mx = jnp.max(g, axis=1, keepdims=True)
        e = jnp.exp(g - mx)
        un = lax.dot_general(e, caug_ref[m], (((1,), (0,)), ((), ())),
                             preferred_element_type=jnp.float32)
        rs = 1.0 / un[:, d:d + 1]
        xr_m = un[:, :d] * rs
        yr_cols.append(xr_m)
        iota = lax.broadcasted_iota(jnp.int32, (TN, K), 1).astype(jnp.float32)
        idx = jnp.min(jnp.where(g >= mx, iota, jnp.float32(K)),
                      axis=1, keepdims=True)
        lab_ref[:, m:m + 1] = idx.astype(jnp.int32)
    yr = jnp.concatenate(yr_cols, axis=1)
    xr_ref[...] = yr
    mr_ref[...] = lax.dot_general(yr, w, (((1,), (1,)), ((), ())),
                                  preferred_element_type=jnp.float32)


def kernel(X, W, C):
    N, D = X.shape
    M, K, d = C.shape
    tn = _TN
    grid = (N // tn,)
    out_shape = (
        jax.ShapeDtypeStruct((N, D), jnp.float32),
        jax.ShapeDtypeStruct((N, D), jnp.float32),
        jax.ShapeDtypeStruct((N, D), jnp.float32),
        jax.ShapeDtypeStruct((N, D), jnp.float32),
        jax.ShapeDtypeStruct((N, M), jnp.int32),
    )
    in_specs = [
        pl.BlockSpec((tn, D), lambda i: (i, 0)),
        pl.BlockSpec((D, D), lambda i: (0, 0)),
        pl.BlockSpec((M, K, d), lambda i: (0, 0, 0)),
    ]
    out_specs = [
        pl.BlockSpec((tn, D), lambda i: (i, 0)),
        pl.BlockSpec((tn, D), lambda i: (i, 0)),
        pl.BlockSpec((tn, D), lambda i: (i, 0)),
        pl.BlockSpec((tn, D), lambda i: (i, 0)),
        pl.BlockSpec((tn, M), lambda i: (i, 0)),
    ]
    xr, xp, mr, mp, lab = pl.pallas_call(
        _fused_body, grid=grid,
        in_specs=in_specs, out_specs=out_specs, out_shape=out_shape,
        scratch_shapes=[
            pltpu.VMEM((M, K), jnp.float32),
            pltpu.VMEM((M, K, d + 1), jnp.float32),
        ],
    )(X, W, C)
    xr3 = jnp.transpose(xr.reshape(N, M, d), (1, 0, 2))
    xp3 = jnp.transpose(xp.reshape(N, M, d), (1, 0, 2))
    return (xr3, xp3, mr, mp, C, lab.T)


# speedup vs baseline: 1.3113x; 1.3113x over previous
import jax
import jax.numpy as jnp
from jax import lax
from jax.experimental import pallas as pl
from jax.experimental.pallas import tpu as pltpu

_TN = 256


def _fused_body(x_ref, w_ref, c_ref,
                xr_ref, xp_ref, mr_ref, mp_ref, lab_ref,
                c2_ref, caug_ref):
    M, K, d = c_ref.shape
    TN = x_ref.shape[0]

    @pl.when(pl.program_id(0) == 0)
    def _init():
        for m in range(M):
            cm = c_ref[m]
            c2_ref[m:m + 1, :] = jnp.sum(cm * cm, axis=1)[None, :]
            caug_ref[m] = jnp.concatenate(
                [cm, jnp.ones((K, 1), jnp.float32)], axis=1)

    x = x_ref[...]
    w = w_ref[...]
    y = lax.dot_general(x, w, (((1,), (0,)), ((), ())),
                        preferred_element_type=jnp.float32)
    xp_ref[...] = y
    mp_ref[...] = lax.dot_general(y, w, (((1,), (1,)), ((), ())),
                                  preferred_element_type=jnp.float32)
    yr_cols = []
    for m in range(M):
        xm = y[:, m * d:(m + 1) * d]
        xs = xm + xm
        cm = c_ref[m]
        xc2 = lax.dot_general(xs, cm, (((1,), (1,)), ((), ())),
                              preferred_element_type=jnp.float32)
        g = xc2 - c2_ref[m:m + 1, :]
        mx = jnp.max(g, axis=1, keepdims=True)
        e = jnp.exp(g - mx)
        un = lax.dot_general(e, caug_ref[m], (((1,), (0,)), ((), ())),
                             preferred_element_type=jnp.float32)
        rs = 1.0 / un[:, d:d + 1]
        xr_m = un[:, :d] * rs
        yr_cols.append(xr_m)
        iota = lax.broadcasted_iota(jnp.int32, (TN, K), 1).astype(jnp.float32)
        idx = jnp.min(jnp.where(g >= mx, iota, jnp.float32(K)),
                      axis=1, keepdims=True)
        lab_ref[:, m:m + 1] = idx.astype(jnp.int32)
    yr = jnp.concatenate(yr_cols, axis=1)
    xr_ref[...] = yr
    mr_ref[...] = lax.dot_general(yr, w, (((1,), (1,)), ((), ())),
                                  preferred_element_type=jnp.float32)


def kernel(X, W, C):
    N, D = X.shape
    M, K, d = C.shape
    tn = _TN
    grid = (N // tn,)
    out_shape = (
        jax.ShapeDtypeStruct((N, D), jnp.float32),
        jax.ShapeDtypeStruct((N, D), jnp.float32),
        jax.ShapeDtypeStruct((N, D), jnp.float32),
        jax.ShapeDtypeStruct((N, D), jnp.float32),
        jax.ShapeDtypeStruct((N, M), jnp.int32),
    )
    in_specs = [
        pl.BlockSpec((tn, D), lambda i: (i, 0)),
        pl.BlockSpec((D, D), lambda i: (0, 0)),
        pl.BlockSpec((M, K, d), lambda i: (0, 0, 0)),
    ]
    out_specs = [
        pl.BlockSpec((tn, D), lambda i: (i, 0)),
        pl.BlockSpec((tn, D), lambda i: (i, 0)),
        pl.BlockSpec((tn, D), lambda i: (i, 0)),
        pl.BlockSpec((tn, D), lambda i: (i, 0)),
        pl.BlockSpec((tn, M), lambda i: (i, 0)),
    ]
    xr, xp, mr, mp, lab = pl.pallas_call(
        _fused_body, grid=grid,
        in_specs=in_specs, out_specs=out_specs, out_shape=out_shape,
        scratch_shapes=[
            pltpu.VMEM((M, K), jnp.float32),
            pltpu.VMEM((M, K, d + 1), jnp.float32),
        ],
    )(X, W, C)
    xr3 = jnp.transpose(xr.reshape(N, M, d), (1, 0, 2))
    xp3 = jnp.transpose(xp.reshape(N, M, d), (1, 0, 2))
    return (xr3, xp3, mr, mp, C, lab.T)
